# Initial kernel scaffold; baseline (speedup 1.0000x reference)
import jax
import jax.numpy as jnp
from jax.experimental import pallas as pl


def kernel(node_features, edges, edge_weights, input_node_indices, params):
    raise NotImplementedError("write your pallas kernel here")


# speedup vs baseline: 4.5429x; 4.5429x over previous
import functools

import jax
import jax.numpy as jnp
import numpy as np
from jax import lax
from jax.experimental import pallas as pl
from jax.experimental.pallas import tpu as pltpu
from jax.experimental.pallas import tpu_sc as plsc

N_NODES = 100000
N_EDGES = 1600000
N_PRED = 10000
NUM_CLASSES = 40
HID = 32
BN_EPS = 1e-3

NC = 2
NS = 16
L = 16

CH = 128
SUPER_E = 1024
SUPERS = 98
E_PER_SUB = SUPER_E * SUPERS
E_PAD = E_PER_SUB * NS
HALF = N_NODES // 2
ACC_R = 50176
R_PER_SUB = ACC_R // NS
TRASH = HALF

GW_CH = 3
PRED_PAD = NC * NS * GW_CH * CH


def _gelu(x):
    return 0.5 * x * (1.0 + lax.erf(x * np.float32(1.0 / np.sqrt(2.0))))


def _fold_ffn(p, pre):
    s1 = p[pre + '_bn1_g'] * (1.0 / np.sqrt(1.0 + BN_EPS))
    b1 = p[pre + '_bn1_b']
    w1 = s1[:, None] * p[pre + '_d1_w']
    c1 = b1 @ p[pre + '_d1_w'] + p[pre + '_d1_b']
    s2 = p[pre + '_bn2_g'] * (1.0 / np.sqrt(1.0 + BN_EPS))
    b2 = p[pre + '_bn2_b']
    w2 = s2[:, None] * p[pre + '_d2_w']
    c2 = b2 @ p[pre + '_d2_w'] + p[pre + '_d2_b']
    return w1, c1[None, :], w2, c2[None, :]


def _conv_index_lists():
    out = []
    for m in range(4):
        pi, pj = m // 2, m % 2
        ii, oo, kk = [], [], []
        for oi in range(8):
            for oj in range(4):
                for c in range(4):
                    for di in range(3):
                        for dj in range(3):
                            i0 = 2 * oi + pi + di - 1
                            j0 = 2 * oj + pj + dj - 1
                            if 0 <= i0 < 16 and 0 <= j0 < 8:
                                ii.append(i0 * 8 + j0)
                                oo.append((oi * 4 + oj) * 4 + c)
                                kk.append((di * 3 + dj) * 4 + c)
        out.append((np.asarray(ii, np.int32), np.asarray(oo, np.int32),
                    np.asarray(kk, np.int32)))
    return out


_CONV_IDX = _conv_index_lists()
_BC_CHAN = np.arange(128, dtype=np.int32) % 4


def _conv_mats(conv_k):
    kflat = conv_k.reshape(36)
    mats = []
    for ii, oo, kk in _CONV_IDX:
        w = jnp.zeros((128, 128), jnp.float32).at[(ii, oo)].add(kflat[kk])
        mats.append(w)
    return jnp.stack(mats)


_BLK = 2000
_NBLK = N_NODES // _BLK


def _full(shape):
    nd = len(shape)
    return pl.BlockSpec(shape, lambda i, _n=nd: (0,) * _n)


def _rowblk(width):
    return pl.BlockSpec((_BLK, width), lambda i: (i, 0))


def _stage_a_body(nf, wc, bc, w1, b1, w2, b2, v1, c1, v2, c2, x0_o, m1_o):
    x = nf[...]
    acc = None
    for m in range(4):
        y = jnp.maximum(jnp.dot(x, wc[m], preferred_element_type=jnp.float32, precision=lax.Precision.HIGHEST)
                        + bc[...], 0.0)
        acc = y if acc is None else jnp.maximum(acc, y)
    h = _gelu(jnp.dot(acc, w1[...], preferred_element_type=jnp.float32, precision=lax.Precision.HIGHEST) + b1[...])
    x0 = _gelu(jnp.dot(h, w2[...], preferred_element_type=jnp.float32, precision=lax.Precision.HIGHEST) + b2[...])
    x0_o[...] = x0
    g = _gelu(jnp.dot(x0, v1[...], preferred_element_type=jnp.float32, precision=lax.Precision.HIGHEST) + c1[...])
    m1_o[...] = _gelu(jnp.dot(g, v2[...], preferred_element_type=jnp.float32, precision=lax.Precision.HIGHEST) + c2[...])


def _stage_a(nf, wc, bc, w1, b1, w2, b2, v1, c1, v2, c2):
    return pl.pallas_call(
        _stage_a_body,
        grid=(_NBLK,),
        in_specs=[_rowblk(128), _full((4, 128, 128)), _full((1, 128)),
                  _full((128, HID)), _full((1, HID)), _full((HID, HID)), _full((1, HID)),
                  _full((HID, HID)), _full((1, HID)), _full((HID, HID)), _full((1, HID))],
        out_specs=[_rowblk(HID), _rowblk(HID)],
        out_shape=[jax.ShapeDtypeStruct((N_NODES, HID), jnp.float32),
                   jax.ShapeDtypeStruct((N_NODES, HID), jnp.float32)],
    )(nf, wc, bc, w1, b1, w2, b2, v1, c1, v2, c2)


def _stage_u_body(x, agg, ua, ub, u1b, u2, u2b, v1, c1, v2, c2,
                  x1_o, m2_o):
    xi = x[...]
    pre = (jnp.dot(xi, ua[...], preferred_element_type=jnp.float32, precision=lax.Precision.HIGHEST)
           + jnp.dot(agg[...], ub[...], preferred_element_type=jnp.float32, precision=lax.Precision.HIGHEST)
           + u1b[...])
    h = _gelu(pre)
    u = _gelu(jnp.dot(h, u2[...], preferred_element_type=jnp.float32, precision=lax.Precision.HIGHEST) + u2b[...])
    u = u * lax.rsqrt(jnp.maximum(jnp.sum(u * u, axis=-1, keepdims=True), 1e-12))
    x1 = u + xi
    x1_o[...] = x1
    g = _gelu(jnp.dot(x1, v1[...], preferred_element_type=jnp.float32, precision=lax.Precision.HIGHEST) + c1[...])
    m2_o[...] = _gelu(jnp.dot(g, v2[...], preferred_element_type=jnp.float32, precision=lax.Precision.HIGHEST) + c2[...])


def _stage_u(x, agg, ua, ub, u1b, u2, u2b, v1, c1, v2, c2):
    return pl.pallas_call(
        _stage_u_body,
        grid=(_NBLK,),
        in_specs=[_rowblk(HID), _rowblk(HID),
                  _full((HID, HID)), _full((HID, HID)), _full((1, HID)),
                  _full((HID, HID)), _full((1, HID)),
                  _full((HID, HID)), _full((1, HID)), _full((HID, HID)), _full((1, HID))],
        out_specs=[_rowblk(HID), _rowblk(HID)],
        out_shape=[jax.ShapeDtypeStruct((N_NODES, HID), jnp.float32),
                   jax.ShapeDtypeStruct((N_NODES, HID), jnp.float32)],
    )(x, agg, ua, ub, u1b, u2, u2b, v1, c1, v2, c2)


def _ewsum_body(ew, o):
    o[...] = jnp.sum(ew[...])[None, None]


def _ewsum(ew):
    ew2 = ew.reshape(1250, 1280)
    return pl.pallas_call(
        _ewsum_body,
        grid=(1,),
        in_specs=[_full((1250, 1280))],
        out_specs=pl.BlockSpec((1, 1), lambda i: (0, 0)),
        out_shape=jax.ShapeDtypeStruct((1, 1), jnp.float32),
    )(ew2)


def _logits_body(emb, w, b, o):
    o[...] = (jnp.dot(emb[...], w[...], preferred_element_type=jnp.float32, precision=lax.Precision.HIGHEST)
              + b[...])


def _logits(emb, w, b):
    return pl.pallas_call(
        _logits_body,
        grid=(1,),
        in_specs=[_full((N_PRED, HID)), _full((HID, NUM_CLASSES)),
                  _full((1, NUM_CLASSES))],
        out_specs=_full((N_PRED, NUM_CLASSES)),
        out_shape=jax.ShapeDtypeStruct((N_PRED, NUM_CLASSES), jnp.float32),
    )(emb, w, b)


_BCAST_DN = lax.GatherDimensionNumbers(
    offset_dims=(), collapsed_slice_dims=(0,), start_index_map=(0,))


def _lane_bcast(vec16, lane):
    idx = jnp.full((L, 1), lane, jnp.int32)
    return lax.gather(vec16, idx, _BCAST_DN, (1,),
                      mode=lax.GatherScatterMode.PROMISE_IN_BOUNDS)


def _agg_sc_body(msg, src, dst, ew, zeros, out,
                 src_v, dst_v, ew_v, dloc_v, rows_v, acc, gsem, ssem):
    c = lax.axis_index("c")
    s = lax.axis_index("s")
    lo = c * HALF

    pltpu.sync_copy(zeros.at[pl.ds(s * R_PER_SUB, R_PER_SUB)],
                    acc.at[pl.ds(s * R_PER_SUB, R_PER_SUB)])
    plsc.subcore_barrier()

    ebase = s * E_PER_SUB

    def slab_copies(t, buf):
        b = ebase + t * SUPER_E
        return [
            pltpu.make_async_copy(src.at[pl.ds(b, SUPER_E)], src_v.at[buf], ssem),
            pltpu.make_async_copy(dst.at[pl.ds(b, SUPER_E)], dst_v.at[buf], ssem),
            pltpu.make_async_copy(ew.at[pl.ds(b, SUPER_E)], ew_v.at[buf], ssem),
        ]

    for d in slab_copies(0, 0):
        d.start()

    def gdesc(tb, j, buf):
        return pltpu.make_async_copy(
            msg.at[src_v.at[tb, pl.ds(j * CH, CH)]], rows_v.at[buf], gsem)

    def super_body(t, carry):
        tb = t % 2
        for d in slab_copies(t, tb):
            d.wait()

        @pl.when(t + 1 < SUPERS)
        def _():
            for d in slab_copies(t + 1, 1 - tb):
                d.start()

        gdesc(tb, 0, 0).start()

        def chunk_body(j, carry2):
            jb = j % 2
            gdesc(tb, j, jb).wait()

            @pl.when(j + 1 < SUPER_E // CH)
            def _():
                gdesc(tb, j + 1, 1 - jb).start()

            for g in range(CH // L):
                d16 = dst_v[tb, pl.ds(j * CH + g * L, L)]
                inr = (d16 >= lo) & (d16 < lo + HALF)
                dloc_v[pl.ds(g * L, L)] = jnp.where(inr, d16 - lo, TRASH)

            def scale_grp(g, _):
                w16 = ew_v[tb, pl.ds(j * CH + g * L, L)]
                for e in range(L):
                    b = _lane_bcast(w16, e)
                    r = g * L + e
                    rows_v[jb, r, pl.ds(0, L)] = rows_v[jb, r, pl.ds(0, L)] * b
                    rows_v[jb, r, pl.ds(L, L)] = rows_v[jb, r, pl.ds(L, L)] * b
                return 0

            lax.fori_loop(0, CH // L, scale_grp, 0)

            pltpu.sync_copy(rows_v.at[jb], acc.at[dloc_v], add=True)
            return 0

        lax.fori_loop(0, SUPER_E // CH, chunk_body, 0)
        return 0

    lax.fori_loop(0, SUPERS, super_body, 0)

    plsc.subcore_barrier()

    @pl.when(s < NS - 1)
    def _():
        pltpu.sync_copy(acc.at[pl.ds(s * R_PER_SUB, R_PER_SUB)],
                        out.at[pl.ds(lo + s * R_PER_SUB, R_PER_SUB)])

    @pl.when(s == NS - 1)
    def _():
        last = HALF - (NS - 1) * R_PER_SUB
        pltpu.sync_copy(acc.at[pl.ds((NS - 1) * R_PER_SUB, last)],
                        out.at[pl.ds(lo + (NS - 1) * R_PER_SUB, last)])


def _agg_sc(msg, src, dst, ew):
    pad = E_PAD - N_EDGES
    srcp = jnp.concatenate([src, jnp.zeros((pad,), jnp.int32)])
    dstp = jnp.concatenate([dst, jnp.zeros((pad,), jnp.int32)])
    ewp = jnp.concatenate([ew, jnp.zeros((pad,), jnp.float32)])
    zeros = jnp.zeros((ACC_R, HID), jnp.float32)
    mesh = plsc.VectorSubcoreMesh(core_axis_name="c", subcore_axis_name="s")
    f = pl.kernel(
        _agg_sc_body,
        out_type=jax.ShapeDtypeStruct((N_NODES, HID), jnp.float32),
        mesh=mesh,
        compiler_params=pltpu.CompilerParams(use_tc_tiling_on_sc=False),
        scratch_types=[
            pltpu.VMEM((2, SUPER_E), jnp.int32),
            pltpu.VMEM((2, SUPER_E), jnp.int32),
            pltpu.VMEM((2, SUPER_E), jnp.float32),
            pltpu.VMEM((CH,), jnp.int32),
            pltpu.VMEM((2, CH, HID), jnp.float32),
            pltpu.VMEM_SHARED((ACC_R, HID), jnp.float32),
            pltpu.SemaphoreType.DMA,
            pltpu.SemaphoreType.DMA,
        ],
    )
    return f(msg, srcp, dstp, ewp, zeros)


def _emb_gather_body(x3, idx, out, idx_v, rows_v, sem):
    c = lax.axis_index("c")
    s = lax.axis_index("s")
    wid = s * NC + c
    pltpu.sync_copy(idx.at[pl.ds(wid * GW_CH, GW_CH)], idx_v)
    descs = [pltpu.make_async_copy(x3.at[idx_v.at[k]], rows_v.at[k], sem)
             for k in range(GW_CH)]
    for d in descs:
        d.start()
    for d in descs:
        d.wait()
    pltpu.sync_copy(rows_v, out.at[pl.ds(wid * GW_CH, GW_CH)])


def _emb_gather(x3, idx):
    idxp = jnp.concatenate([idx, jnp.zeros((PRED_PAD - N_PRED,), jnp.int32)])
    idxp = idxp.reshape(NC * NS * GW_CH, CH)
    mesh = plsc.VectorSubcoreMesh(core_axis_name="c", subcore_axis_name="s")
    f = pl.kernel(
        _emb_gather_body,
        out_type=jax.ShapeDtypeStruct((NC * NS * GW_CH, CH, HID), jnp.float32),
        mesh=mesh,
        compiler_params=pltpu.CompilerParams(use_tc_tiling_on_sc=False),
        scratch_types=[
            pltpu.VMEM((GW_CH, CH), jnp.int32),
            pltpu.VMEM((GW_CH, CH, HID), jnp.float32),
            pltpu.SemaphoreType.DMA,
        ],
    )
    return f(x3, idxp).reshape(PRED_PAD, HID)[:N_PRED]


def kernel(node_features, edges, edge_weights, input_node_indices, params):
    p = params
    nf = node_features.reshape(N_NODES, 128)
    src = edges[1].astype(jnp.int32)
    dst = edges[0].astype(jnp.int32)
    ew = edge_weights
    idx = input_node_indices.astype(jnp.int32)

    wc = _conv_mats(p['conv_k'])
    bc = jnp.take(p['conv_b'], _BC_CHAN)[None, :]
    pre_w1, pre_b1, pre_w2, pre_b2 = _fold_ffn(p, 'pre')
    c1p = _fold_ffn(p, 'c1_p')
    c1u = _fold_ffn(p, 'c1_u')
    c2p = _fold_ffn(p, 'c2_p')
    c2u = _fold_ffn(p, 'c2_u')
    post = _fold_ffn(p, 'post')

    s_ew = _ewsum(ew)[0, 0]
    inv_s = 1.0 / s_ew

    x0, msg1 = _stage_a(nf, wc, bc, pre_w1, pre_b1, pre_w2, pre_b2,
                        c1p[0], c1p[1], c1p[2], c1p[3])

    agg1 = _agg_sc(msg1, src, dst, ew)
    u1a, u1b_agg = c1u[0][:HID], c1u[0][HID:] * inv_s
    x1, msg2 = _stage_u(x0, agg1, u1a, u1b_agg, c1u[1], c1u[2], c1u[3],
                        c2p[0], c2p[1], c2p[2], c2p[3])

    agg2 = _agg_sc(msg2, src, dst, ew)
    u2a, u2b_agg = c2u[0][:HID], c2u[0][HID:] * inv_s
    x2, x3 = _stage_u(x1, agg2, u2a, u2b_agg, c2u[1], c2u[2], c2u[3],
                      post[0], post[1], post[2], post[3])
    del x2

    emb = _emb_gather(x3, idx)
    return _logits(emb, p['log_w'], p['log_b'][None, :])


# speedup vs baseline: 5.9928x; 1.3192x over previous
import functools

import jax
import jax.numpy as jnp
import numpy as np
from jax import lax
from jax.experimental import pallas as pl
from jax.experimental.pallas import tpu as pltpu
from jax.experimental.pallas import tpu_sc as plsc

N_NODES = 100000
N_EDGES = 1600000
N_PRED = 10000
NUM_CLASSES = 40
HID = 32
BN_EPS = 1e-3

NC = 2
NS = 16
L = 16

CH = 128
SUPER_E = 1024
SUPERS = 98
E_PER_SUB = SUPER_E * SUPERS
E_PAD = E_PER_SUB * NS
HALF = N_NODES // 2
ACC_R = 50176
R_PER_SUB = ACC_R // NS
TRASH = HALF

GW_CH = 3
PRED_PAD = NC * NS * GW_CH * CH


def _gelu(x):
    return 0.5 * x * (1.0 + lax.erf(x * np.float32(1.0 / np.sqrt(2.0))))


def _fold_ffn(p, pre):
    s1 = p[pre + '_bn1_g'] * (1.0 / np.sqrt(1.0 + BN_EPS))
    b1 = p[pre + '_bn1_b']
    w1 = s1[:, None] * p[pre + '_d1_w']
    c1 = b1 @ p[pre + '_d1_w'] + p[pre + '_d1_b']
    s2 = p[pre + '_bn2_g'] * (1.0 / np.sqrt(1.0 + BN_EPS))
    b2 = p[pre + '_bn2_b']
    w2 = s2[:, None] * p[pre + '_d2_w']
    c2 = b2 @ p[pre + '_d2_w'] + p[pre + '_d2_b']
    return w1, c1[None, :], w2, c2[None, :]


def _conv_index_lists():
    out = []
    for m in range(4):
        pi, pj = m // 2, m % 2
        ii, oo, kk = [], [], []
        for oi in range(8):
            for oj in range(4):
                for c in range(4):
                    for di in range(3):
                        for dj in range(3):
                            i0 = 2 * oi + pi + di - 1
                            j0 = 2 * oj + pj + dj - 1
                            if 0 <= i0 < 16 and 0 <= j0 < 8:
                                ii.append(i0 * 8 + j0)
                                oo.append((oi * 4 + oj) * 4 + c)
                                kk.append((di * 3 + dj) * 4 + c)
        out.append((np.asarray(ii, np.int32), np.asarray(oo, np.int32),
                    np.asarray(kk, np.int32)))
    return out


_CONV_IDX = _conv_index_lists()
_BC_CHAN = np.arange(128, dtype=np.int32) % 4


def _conv_mats(conv_k):
    kflat = conv_k.reshape(36)
    mats = []
    for ii, oo, kk in _CONV_IDX:
        w = jnp.zeros((128, 128), jnp.float32).at[(ii, oo)].add(kflat[kk])
        mats.append(w)
    return jnp.stack(mats)


_BLK = 5000
_NBLK = N_NODES // _BLK


def _full(shape):
    nd = len(shape)
    return pl.BlockSpec(shape, lambda i, _n=nd: (0,) * _n)


def _rowblk(width):
    return pl.BlockSpec((_BLK, width), lambda i: (i, 0))


def _stage_a_body(nf, wc, bc, w1, b1, w2, b2, v1, c1, v2, c2, x0_o, m1_o):
    x = nf[...]
    acc = None
    for m in range(4):
        y = jnp.maximum(jnp.dot(x, wc[m], preferred_element_type=jnp.float32)
                        + bc[...], 0.0)
        acc = y if acc is None else jnp.maximum(acc, y)
    h = _gelu(jnp.dot(acc, w1[...], preferred_element_type=jnp.float32) + b1[...])
    x0 = _gelu(jnp.dot(h, w2[...], preferred_element_type=jnp.float32) + b2[...])
    x0_o[...] = x0
    g = _gelu(jnp.dot(x0, v1[...], preferred_element_type=jnp.float32) + c1[...])
    m1_o[...] = _gelu(jnp.dot(g, v2[...], preferred_element_type=jnp.float32) + c2[...])


def _stage_a(nf, wc, bc, w1, b1, w2, b2, v1, c1, v2, c2):
    return pl.pallas_call(
        _stage_a_body,
        grid=(_NBLK,),
        in_specs=[_rowblk(128), _full((4, 128, 128)), _full((1, 128)),
                  _full((128, HID)), _full((1, HID)), _full((HID, HID)), _full((1, HID)),
                  _full((HID, HID)), _full((1, HID)), _full((HID, HID)), _full((1, HID))],
        out_specs=[_rowblk(HID), _rowblk(HID)],
        out_shape=[jax.ShapeDtypeStruct((N_NODES, HID), jnp.float32),
                   jax.ShapeDtypeStruct((N_NODES, HID), jnp.float32)],
    )(nf, wc, bc, w1, b1, w2, b2, v1, c1, v2, c2)


def _stage_u_body(x, agg, ua, ub, u1b, u2, u2b, v1, c1, v2, c2,
                  x1_o, m2_o):
    xi = x[...]
    pre = (jnp.dot(xi, ua[...], preferred_element_type=jnp.float32)
           + jnp.dot(agg[...], ub[...], preferred_element_type=jnp.float32)
           + u1b[...])
    h = _gelu(pre)
    u = _gelu(jnp.dot(h, u2[...], preferred_element_type=jnp.float32) + u2b[...])
    u = u * lax.rsqrt(jnp.maximum(jnp.sum(u * u, axis=-1, keepdims=True), 1e-12))
    x1 = u + xi
    x1_o[...] = x1
    g = _gelu(jnp.dot(x1, v1[...], preferred_element_type=jnp.float32) + c1[...])
    m2_o[...] = _gelu(jnp.dot(g, v2[...], preferred_element_type=jnp.float32) + c2[...])


def _stage_u(x, agg, ua, ub, u1b, u2, u2b, v1, c1, v2, c2):
    return pl.pallas_call(
        _stage_u_body,
        grid=(_NBLK,),
        in_specs=[_rowblk(HID), _rowblk(HID),
                  _full((HID, HID)), _full((HID, HID)), _full((1, HID)),
                  _full((HID, HID)), _full((1, HID)),
                  _full((HID, HID)), _full((1, HID)), _full((HID, HID)), _full((1, HID))],
        out_specs=[_rowblk(HID), _rowblk(HID)],
        out_shape=[jax.ShapeDtypeStruct((N_NODES, HID), jnp.float32),
                   jax.ShapeDtypeStruct((N_NODES, HID), jnp.float32)],
    )(x, agg, ua, ub, u1b, u2, u2b, v1, c1, v2, c2)


def _ewsum_body(ew, o):
    o[...] = jnp.sum(ew[...])[None, None]


def _ewsum(ew):
    ew2 = ew.reshape(1250, 1280)
    return pl.pallas_call(
        _ewsum_body,
        grid=(1,),
        in_specs=[_full((1250, 1280))],
        out_specs=pl.BlockSpec((1, 1), lambda i: (0, 0)),
        out_shape=jax.ShapeDtypeStruct((1, 1), jnp.float32),
    )(ew2)


def _logits_body(emb, w, b, o):
    o[...] = (jnp.dot(emb[...], w[...], preferred_element_type=jnp.float32)
              + b[...])


def _logits(emb, w, b):
    return pl.pallas_call(
        _logits_body,
        grid=(1,),
        in_specs=[_full((N_PRED, HID)), _full((HID, NUM_CLASSES)),
                  _full((1, NUM_CLASSES))],
        out_specs=_full((N_PRED, NUM_CLASSES)),
        out_shape=jax.ShapeDtypeStruct((N_PRED, NUM_CLASSES), jnp.float32),
    )(emb, w, b)


_BCAST_DN = lax.GatherDimensionNumbers(
    offset_dims=(), collapsed_slice_dims=(0,), start_index_map=(0,))


def _lane_bcast(vec16, lane):
    idx = jnp.full((L, 1), lane, jnp.int32)
    return lax.gather(vec16, idx, _BCAST_DN, (1,),
                      mode=lax.GatherScatterMode.PROMISE_IN_BOUNDS)


def _agg_sc_body(msg, src, dst, ew, zeros, out,
                 src_v, dst_v, ew_v, dloc_v, rows_v, acc, gsem, ssem, vsem):
    c = lax.axis_index("c")
    s = lax.axis_index("s")
    lo = c * HALF

    pltpu.sync_copy(zeros.at[pl.ds(s * R_PER_SUB, R_PER_SUB)],
                    acc.at[pl.ds(s * R_PER_SUB, R_PER_SUB)])
    plsc.subcore_barrier()

    ebase = s * E_PER_SUB
    n_chunks = E_PER_SUB // CH

    def slab_copies(t, buf):
        b = ebase + t * SUPER_E
        return [
            pltpu.make_async_copy(src.at[pl.ds(b, SUPER_E)], src_v.at[buf], ssem),
            pltpu.make_async_copy(dst.at[pl.ds(b, SUPER_E)], dst_v.at[buf], ssem),
            pltpu.make_async_copy(ew.at[pl.ds(b, SUPER_E)], ew_v.at[buf], ssem),
        ]

    def gdesc(k, buf):
        t = k // (SUPER_E // CH)
        j = k % (SUPER_E // CH)
        return pltpu.make_async_copy(
            msg.at[src_v.at[t % 2, pl.ds(j * CH, CH)]], rows_v.at[buf], gsem)

    def sdesc_start(kb):
        pltpu.async_copy(rows_v.at[kb % 3], acc.at[dloc_v.at[kb % 2]], vsem,
                         add=True)

    def sdesc_wait(kb):
        pltpu.make_async_copy(rows_v.at[kb % 3],
                              acc.at[dloc_v.at[kb % 2]], vsem).wait()

    for d in slab_copies(0, 0):
        d.start()
    for d in slab_copies(0, 0):
        d.wait()
    for d in slab_copies(1, 1):
        d.start()
    gdesc(0, 0).start()

    def chunk_body(k, carry):
        kb3 = k % 3
        kb2 = k % 2
        tb = (k // (SUPER_E // CH)) % 2

        gdesc(k, kb3).wait()

        @pl.when(jnp.logical_and(k >= 8, k % 8 == 0))
        def _():
            t = k // 8

            @pl.when(t + 1 < SUPERS)
            def _():
                for d in slab_copies(t + 1, (t + 1) % 2):
                    d.start()

        @pl.when(k >= 2)
        def _():
            sdesc_wait(k - 2)

        @pl.when(k + 1 < n_chunks)
        def _():
            @pl.when((k + 1) % 8 == 0)
            def _():
                for d in slab_copies((k + 1) // 8, ((k + 1) // 8) % 2):
                    d.wait()

            gdesc(k + 1, (k + 1) % 3).start()

        j8 = k % (SUPER_E // CH)

        for g in range(CH // L):
            d16 = dst_v[tb, pl.ds(j8 * CH + g * L, L)]
            inr = (d16 >= lo) & (d16 < lo + HALF)
            dloc_v[kb2, pl.ds(g * L, L)] = jnp.where(inr, d16 - lo, TRASH)

        def scale_grp(g, _):
            w16 = ew_v[tb, pl.ds(j8 * CH + g * L, L)]
            for e in range(L):
                b = _lane_bcast(w16, e)
                r = g * L + e
                rows_v[kb3, r, pl.ds(0, L)] = rows_v[kb3, r, pl.ds(0, L)] * b
                rows_v[kb3, r, pl.ds(L, L)] = rows_v[kb3, r, pl.ds(L, L)] * b
            return 0

        lax.fori_loop(0, CH // L, scale_grp, 0)

        sdesc_start(k)
        return 0

    lax.fori_loop(0, n_chunks, chunk_body, 0)

    sdesc_wait(n_chunks - 2)
    sdesc_wait(n_chunks - 1)

    plsc.subcore_barrier()

    @pl.when(s < NS - 1)
    def _():
        pltpu.sync_copy(acc.at[pl.ds(s * R_PER_SUB, R_PER_SUB)],
                        out.at[pl.ds(lo + s * R_PER_SUB, R_PER_SUB)])

    @pl.when(s == NS - 1)
    def _():
        last = HALF - (NS - 1) * R_PER_SUB
        pltpu.sync_copy(acc.at[pl.ds((NS - 1) * R_PER_SUB, last)],
                        out.at[pl.ds(lo + (NS - 1) * R_PER_SUB, last)])


def _agg_sc(msg, srcp, dstp, ewp):
    zeros = jnp.zeros((ACC_R, HID), jnp.float32)
    mesh = plsc.VectorSubcoreMesh(core_axis_name="c", subcore_axis_name="s")
    f = pl.kernel(
        _agg_sc_body,
        out_type=jax.ShapeDtypeStruct((N_NODES, HID), jnp.float32),
        mesh=mesh,
        compiler_params=pltpu.CompilerParams(use_tc_tiling_on_sc=False),
        scratch_types=[
            pltpu.VMEM((2, SUPER_E), jnp.int32),
            pltpu.VMEM((2, SUPER_E), jnp.int32),
            pltpu.VMEM((2, SUPER_E), jnp.float32),
            pltpu.VMEM((2, CH), jnp.int32),
            pltpu.VMEM((3, CH, HID), jnp.float32),
            pltpu.VMEM_SHARED((ACC_R, HID), jnp.float32),
            pltpu.SemaphoreType.DMA,
            pltpu.SemaphoreType.DMA,
            pltpu.SemaphoreType.DMA,
        ],
    )
    return f(msg, srcp, dstp, ewp, zeros)


def _emb_gather_body(x3, idx, out, idx_v, rows_v, sem):
    c = lax.axis_index("c")
    s = lax.axis_index("s")
    wid = s * NC + c
    pltpu.sync_copy(idx.at[pl.ds(wid * GW_CH, GW_CH)], idx_v)
    descs = [pltpu.make_async_copy(x3.at[idx_v.at[k]], rows_v.at[k], sem)
             for k in range(GW_CH)]
    for d in descs:
        d.start()
    for d in descs:
        d.wait()
    pltpu.sync_copy(rows_v, out.at[pl.ds(wid * GW_CH, GW_CH)])


def _emb_gather(x3, idx):
    idxp = jnp.concatenate([idx, jnp.zeros((PRED_PAD - N_PRED,), jnp.int32)])
    idxp = idxp.reshape(NC * NS * GW_CH, CH)
    mesh = plsc.VectorSubcoreMesh(core_axis_name="c", subcore_axis_name="s")
    f = pl.kernel(
        _emb_gather_body,
        out_type=jax.ShapeDtypeStruct((NC * NS * GW_CH, CH, HID), jnp.float32),
        mesh=mesh,
        compiler_params=pltpu.CompilerParams(use_tc_tiling_on_sc=False),
        scratch_types=[
            pltpu.VMEM((GW_CH, CH), jnp.int32),
            pltpu.VMEM((GW_CH, CH, HID), jnp.float32),
            pltpu.SemaphoreType.DMA,
        ],
    )
    return f(x3, idxp).reshape(PRED_PAD, HID)[:N_PRED]


def kernel(node_features, edges, edge_weights, input_node_indices, params):
    p = params
    nf = node_features.reshape(N_NODES, 128)
    pad = E_PAD - N_EDGES
    src = jnp.concatenate([edges[1].astype(jnp.int32), jnp.zeros((pad,), jnp.int32)])
    dst = jnp.concatenate([edges[0].astype(jnp.int32), jnp.zeros((pad,), jnp.int32)])
    ew = jnp.concatenate([edge_weights, jnp.zeros((pad,), jnp.float32)])
    idx = input_node_indices.astype(jnp.int32)

    wc = _conv_mats(p['conv_k'])
    bc = jnp.take(p['conv_b'], _BC_CHAN)[None, :]
    pre_w1, pre_b1, pre_w2, pre_b2 = _fold_ffn(p, 'pre')
    c1p = _fold_ffn(p, 'c1_p')
    c1u = _fold_ffn(p, 'c1_u')
    c2p = _fold_ffn(p, 'c2_p')
    c2u = _fold_ffn(p, 'c2_u')
    post = _fold_ffn(p, 'post')

    s_ew = _ewsum(edge_weights)[0, 0]
    inv_s = 1.0 / s_ew

    x0, msg1 = _stage_a(nf, wc, bc, pre_w1, pre_b1, pre_w2, pre_b2,
                        c1p[0], c1p[1], c1p[2], c1p[3])

    agg1 = _agg_sc(msg1, src, dst, ew)
    u1a, u1b_agg = c1u[0][:HID], c1u[0][HID:] * inv_s
    x1, msg2 = _stage_u(x0, agg1, u1a, u1b_agg, c1u[1], c1u[2], c1u[3],
                        c2p[0], c2p[1], c2p[2], c2p[3])

    agg2 = _agg_sc(msg2, src, dst, ew)
    u2a, u2b_agg = c2u[0][:HID], c2u[0][HID:] * inv_s
    x2, x3 = _stage_u(x1, agg2, u2a, u2b_agg, c2u[1], c2u[2], c2u[3],
                      post[0], post[1], post[2], post[3])
    del x2

    emb = _emb_gather(x3, idx)
    return _logits(emb, p['log_w'], p['log_b'][None, :])
